# baseline (speedup 1.0000x reference)
import functools

import jax
import jax.numpy as jnp
from jax import lax
from jax.experimental import pallas as pl
from jax.experimental.pallas import tpu as pltpu
from jax.experimental.pallas import tpu_sc as plsc

N = 10000
D = 128
E = 320000

NC = 2
NS = 16
NW = NC * NS

CH = 128
TOT_CHUNKS = 2560
E_PAD = TOT_CHUNKS * CH
KPT = TOT_CHUNKS // NS
assert KPT % 2 == 0

N_PAD = 10240
RPS = N_PAD // NS

def _i0():
    return jnp.int32(0)


BM = 1000
BD_I = 1024
BD_J = 1024


def _sc_scatter_body(h_hbm, src_hbm, dst_hbm, zero_hbm, out0,
                     srcv0, srcv1, dstv0, dstv1, rows0, rows1, acc_sh,
                     sis0, sis1, sid0, sid1, sg0, sg1):
    cid = lax.axis_index("c")
    sid = lax.axis_index("s")

    @pl.when(cid == 1)
    def _all_work():
        _sc_scatter_work(h_hbm, src_hbm, dst_hbm, zero_hbm, out0,
                         srcv0, srcv1, dstv0, dstv1, rows0, rows1, acc_sh,
                         sis0, sis1, sid0, sid1, sg0, sg1, sid)


def _sc_scatter_work(h_hbm, src_hbm, dst_hbm, zero_hbm, out0,
                     srcv0, srcv1, dstv0, dstv1, rows0, rows1, acc_sh,
                     sis0, sis1, sid0, sid1, sg0, sg1, sid):
    kk = jnp.int32(KPT)
    base = sid * jnp.int32(KPT)

    zsl = pl.ds(sid * RPS, RPS)
    pltpu.sync_copy(zero_hbm.at[zsl], acc_sh.at[zsl])

    plsc.subcore_barrier()

    def edge_sl(c):
        return pl.ds((base + c) * jnp.int32(CH), CH)

    si = (sis0, sis1)
    sd = (sid0, sid1)
    sg = (sg0, sg1)
    srcb = (srcv0, srcv1)
    dstb = (dstv0, dstv1)
    rows = (rows0, rows1)

    def idx_dma(c, p):
        pltpu.async_copy(src_hbm.at[edge_sl(c)], srcb[p], si[p])
        pltpu.async_copy(dst_hbm.at[edge_sl(c)], dstb[p], sd[p])

    def wait_idx(p):
        z = pl.ds(jnp.int32(0), CH)
        pltpu.make_async_copy(src_hbm.at[z], srcb[p], si[p]).wait()
        pltpu.make_async_copy(dst_hbm.at[z], dstb[p], sd[p]).wait()

    def gather(c, p):
        pltpu.async_copy(h_hbm.at[srcb[p]], rows[p], sg[p])

    def wait_gather(p):
        pltpu.make_async_copy(h_hbm.at[srcb[p]], rows[p], sg[p]).wait()

    def scatter(p):
        pltpu.sync_copy(rows[p], acc_sh.at[dstb[p]], add=True)

    idx_dma(jnp.int32(0), 0)
    idx_dma(jnp.int32(1), 1)
    wait_idx(0)
    gather(jnp.int32(0), 0)

    def step(i, carry):
        c = i * jnp.int32(2)
        wait_idx(1)
        gather(c + 1, 1)
        wait_gather(0)
        scatter(0)
        idx_dma(c + 2, 0)
        wait_idx(0)
        gather(c + 2, 0)
        wait_gather(1)
        scatter(1)
        idx_dma(c + 3, 1)
        return carry

    lax.fori_loop(jnp.int32(0), (kk - 2) // 2, step, jnp.int32(0))
    wait_idx(1)
    gather(kk - 1, 1)
    wait_gather(0)
    scatter(0)
    wait_gather(1)
    scatter(1)

    plsc.subcore_barrier()

    osl = pl.ds(sid * RPS, RPS)
    pltpu.sync_copy(acc_sh.at[osl], out0.at[osl])


def _sc_scatter(h, src, dst, zeros):
    mesh = plsc.VectorSubcoreMesh(core_axis_name="c", subcore_axis_name="s")
    f = functools.partial(
        pl.kernel,
        out_type=jax.ShapeDtypeStruct((N_PAD, D), jnp.float32),
        mesh=mesh,
        scratch_types=[
            pltpu.VMEM((CH,), jnp.int32),
            pltpu.VMEM((CH,), jnp.int32),
            pltpu.VMEM((CH,), jnp.int32),
            pltpu.VMEM((CH,), jnp.int32),
            pltpu.VMEM((CH, D), jnp.float32),
            pltpu.VMEM((CH, D), jnp.float32),
            pltpu.VMEM_SHARED((N_PAD, D), jnp.float32),
            pltpu.SemaphoreType.DMA,
            pltpu.SemaphoreType.DMA,
            pltpu.SemaphoreType.DMA,
            pltpu.SemaphoreType.DMA,
            pltpu.SemaphoreType.DMA,
            pltpu.SemaphoreType.DMA,
        ],
    )(_sc_scatter_body)
    return f(h, src, dst, zeros)


def _conv_body(p_ref, x_ref, wr_ref, wt_ref, b_ref, o_ref, *, relu):
    agg = p_ref[...]
    y = (
        jnp.dot(agg, wr_ref[...], preferred_element_type=jnp.float32,
                precision=lax.Precision.HIGHEST)
        + b_ref[...]
        + jnp.dot(x_ref[...], wt_ref[...], preferred_element_type=jnp.float32,
                  precision=lax.Precision.HIGHEST)
    )
    if relu:
        y = jnp.maximum(y, 0.0)
    o_ref[...] = y


def _conv_tc(p, x, w_rel_t, w_root_t, b2d, relu):
    grid = (N // BM,)
    return pl.pallas_call(
        functools.partial(_conv_body, relu=relu),
        grid=grid,
        in_specs=[
            pl.BlockSpec((BM, D), lambda i: (i, _i0())),
            pl.BlockSpec((BM, D), lambda i: (i, _i0())),
            pl.BlockSpec((D, D), lambda i: (_i0(), _i0())),
            pl.BlockSpec((D, D), lambda i: (_i0(), _i0())),
            pl.BlockSpec((1, D), lambda i: (_i0(), _i0())),
        ],
        out_specs=pl.BlockSpec((BM, D), lambda i: (i, _i0())),
        out_shape=jax.ShapeDtypeStruct((N, D), jnp.float32),
    )(p, x, w_rel_t, w_root_t, b2d)


def _decoder_body(zi_ref, zj_ref, o_ref):
    zi = zi_ref[...]
    zj = zj_ref[...]
    zi_hi = zi.astype(jnp.bfloat16)
    zi_lo = (zi - zi_hi.astype(jnp.float32)).astype(jnp.bfloat16)
    zj_hi = zj.astype(jnp.bfloat16)
    zj_lo = (zj - zj_hi.astype(jnp.float32)).astype(jnp.bfloat16)
    dn = (((1,), (1,)), ((), ()))
    logits = lax.dot_general(zi_hi, zj_hi, dn,
                             preferred_element_type=jnp.float32)
    logits += lax.dot_general(zi_hi, zj_lo, dn,
                              preferred_element_type=jnp.float32)
    logits += lax.dot_general(zi_lo, zj_hi, dn,
                              preferred_element_type=jnp.float32)
    o_ref[...] = 1.0 / (1.0 + jnp.exp(-logits))


def _decoder_tc(z):
    grid = (pl.cdiv(N, BD_I), pl.cdiv(N, BD_J))
    return pl.pallas_call(
        _decoder_body,
        grid=grid,
        in_specs=[
            pl.BlockSpec((BD_I, D), lambda i, j: (i, _i0())),
            pl.BlockSpec((BD_J, D), lambda i, j: (j, _i0())),
        ],
        out_specs=pl.BlockSpec((BD_I, BD_J), lambda i, j: (i, j)),
        out_shape=jax.ShapeDtypeStruct((N, N), jnp.float32),
        compiler_params=pltpu.CompilerParams(
            dimension_semantics=("parallel", "parallel"),
        ),
    )(z, z)


def kernel(x, adj, W_rel, b_rel, W_root):
    x = x.astype(jnp.float32)
    src = adj[0].astype(jnp.int32)
    dst = adj[1].astype(jnp.int32)
    pad = E_PAD - E
    src = jnp.concatenate([src, jnp.zeros((pad,), jnp.int32)])
    dst = jnp.concatenate([dst, jnp.full((pad,), N_PAD - 1, jnp.int32)])

    zeros = jnp.zeros((N_PAD, D), jnp.float32)
    w_rel_t = W_rel.astype(jnp.float32).T
    w_root_t = W_root.astype(jnp.float32).T
    b2d = b_rel.astype(jnp.float32).reshape(1, D)

    p = _sc_scatter(x, src, dst, zeros)
    h1 = _conv_tc(p, x, w_rel_t, w_root_t, b2d, relu=True)
    q = _sc_scatter(h1, src, dst, zeros)
    x2 = _conv_tc(q, h1, w_rel_t, w_root_t, b2d, relu=False)
    z_pad = jnp.pad(x2, ((0, N_PAD - N), (0, 0)))
    A = _decoder_tc(z_pad)
    return (A, x2)


# speedup vs baseline: 1.1960x; 1.1960x over previous
import functools

import jax
import jax.numpy as jnp
from jax import lax
from jax.experimental import pallas as pl
from jax.experimental.pallas import tpu as pltpu
from jax.experimental.pallas import tpu_sc as plsc

N = 10000
D = 128
E = 320000

NC = 2
NS = 16
NW = NC * NS

CH = 128
TOT_CHUNKS = 2560
E_PAD = TOT_CHUNKS * CH
K0 = 132
K1 = 28
assert NS * (K0 + K1) == TOT_CHUNKS and K0 % 2 == 0 and K1 % 2 == 0

N_PAD = 10240
RPS = N_PAD // NS

def _i0():
    return jnp.int32(0)


BM = 1000
BD_I = 1024
BD_J = 1024


def _sc_scatter_body(h_hbm, src_hbm, dst_hbm, out0, out1,
                     srcv0, srcv1, dstv0, dstv1, rows0, rows1, acc_sh,
                     sis0, sis1, sid0, sid1, sg0, sg1):
    cid = lax.axis_index("c")
    sid = lax.axis_index("s")

    kk = jnp.where(cid == 0, jnp.int32(K0), jnp.int32(K1))
    base = jnp.where(cid == 0, sid * jnp.int32(K0),
                     jnp.int32(NS * K0) + sid * jnp.int32(K1))

    zv = jnp.zeros((16,), jnp.float32)

    def zrow(r, carry):
        for c8 in range(D // 16):
            rows0[r, pl.ds(jnp.int32(c8 * 16), 16)] = zv
        return carry

    lax.fori_loop(jnp.int32(0), jnp.int32(CH), zrow, jnp.int32(0))
    for s5 in range(RPS // CH):
        pltpu.sync_copy(
            rows0, acc_sh.at[pl.ds(sid * RPS + jnp.int32(s5 * CH), CH)])

    plsc.subcore_barrier()

    def edge_sl(c):
        return pl.ds((base + c) * jnp.int32(CH), CH)

    si = (sis0, sis1)
    sd = (sid0, sid1)
    sg = (sg0, sg1)
    srcb = (srcv0, srcv1)
    dstb = (dstv0, dstv1)
    rows = (rows0, rows1)

    def idx_dma(c, p):
        pltpu.async_copy(src_hbm.at[edge_sl(c)], srcb[p], si[p])
        pltpu.async_copy(dst_hbm.at[edge_sl(c)], dstb[p], sd[p])

    def wait_idx(p):
        z = pl.ds(jnp.int32(0), CH)
        pltpu.make_async_copy(src_hbm.at[z], srcb[p], si[p]).wait()
        pltpu.make_async_copy(dst_hbm.at[z], dstb[p], sd[p]).wait()

    def gather(c, p):
        pltpu.async_copy(h_hbm.at[srcb[p]], rows[p], sg[p])

    def wait_gather(p):
        pltpu.make_async_copy(h_hbm.at[srcb[p]], rows[p], sg[p]).wait()

    def scatter(p):
        pltpu.sync_copy(rows[p], acc_sh.at[dstb[p]], add=True)

    idx_dma(jnp.int32(0), 0)
    idx_dma(jnp.int32(1), 1)
    wait_idx(0)
    gather(jnp.int32(0), 0)

    def step(i, carry):
        c = i * jnp.int32(2)
        wait_idx(1)
        gather(c + 1, 1)
        wait_gather(0)
        scatter(0)
        idx_dma(c + 2, 0)
        wait_idx(0)
        gather(c + 2, 0)
        wait_gather(1)
        scatter(1)
        idx_dma(c + 3, 1)
        return carry

    lax.fori_loop(jnp.int32(0), (kk - 2) // 2, step, jnp.int32(0))
    wait_idx(1)
    gather(kk - 1, 1)
    wait_gather(0)
    scatter(0)
    wait_gather(1)
    scatter(1)

    plsc.subcore_barrier()

    osl = pl.ds(sid * RPS, RPS)

    @pl.when(cid == 0)
    def _():
        pltpu.sync_copy(acc_sh.at[osl], out0.at[osl])

    @pl.when(cid == 1)
    def _():
        pltpu.sync_copy(acc_sh.at[osl], out1.at[osl])


def _sc_scatter(h, src, dst):
    mesh = plsc.VectorSubcoreMesh(core_axis_name="c", subcore_axis_name="s")
    f = functools.partial(
        pl.kernel,
        out_type=(
            jax.ShapeDtypeStruct((N_PAD, D), jnp.float32),
            jax.ShapeDtypeStruct((N_PAD, D), jnp.float32),
        ),
        mesh=mesh,
        scratch_types=[
            pltpu.VMEM((CH,), jnp.int32),
            pltpu.VMEM((CH,), jnp.int32),
            pltpu.VMEM((CH,), jnp.int32),
            pltpu.VMEM((CH,), jnp.int32),
            pltpu.VMEM((CH, D), jnp.float32),
            pltpu.VMEM((CH, D), jnp.float32),
            pltpu.VMEM_SHARED((N_PAD, D), jnp.float32),
            pltpu.SemaphoreType.DMA,
            pltpu.SemaphoreType.DMA,
            pltpu.SemaphoreType.DMA,
            pltpu.SemaphoreType.DMA,
            pltpu.SemaphoreType.DMA,
            pltpu.SemaphoreType.DMA,
        ],
    )(_sc_scatter_body)
    return f(h, src, dst)


def _conv_body(p0_ref, p1_ref, x_ref, wr_ref, wt_ref, b_ref, o_ref, *, relu):
    agg = p0_ref[...] + p1_ref[...]
    y = (
        jnp.dot(agg, wr_ref[...], preferred_element_type=jnp.float32,
                precision=lax.Precision.HIGHEST)
        + b_ref[...]
        + jnp.dot(x_ref[...], wt_ref[...], preferred_element_type=jnp.float32,
                  precision=lax.Precision.HIGHEST)
    )
    if relu:
        y = jnp.maximum(y, 0.0)
    o_ref[...] = y


def _conv_tc(p0, p1, x, w_rel_t, w_root_t, b2d, relu):
    grid = (N // BM,)
    return pl.pallas_call(
        functools.partial(_conv_body, relu=relu),
        grid=grid,
        in_specs=[
            pl.BlockSpec((BM, D), lambda i: (i, _i0())),
            pl.BlockSpec((BM, D), lambda i: (i, _i0())),
            pl.BlockSpec((BM, D), lambda i: (i, _i0())),
            pl.BlockSpec((D, D), lambda i: (_i0(), _i0())),
            pl.BlockSpec((D, D), lambda i: (_i0(), _i0())),
            pl.BlockSpec((1, D), lambda i: (_i0(), _i0())),
        ],
        out_specs=pl.BlockSpec((BM, D), lambda i: (i, _i0())),
        out_shape=jax.ShapeDtypeStruct((N, D), jnp.float32),
    )(p0, p1, x, w_rel_t, w_root_t, b2d)


def _decoder_body(zi_ref, zj_ref, o_ref):
    zi = zi_ref[...]
    zj = zj_ref[...]
    zi_hi = zi.astype(jnp.bfloat16)
    zi_lo = (zi - zi_hi.astype(jnp.float32)).astype(jnp.bfloat16)
    zj_hi = zj.astype(jnp.bfloat16)
    zj_lo = (zj - zj_hi.astype(jnp.float32)).astype(jnp.bfloat16)
    dn = (((1,), (1,)), ((), ()))
    logits = lax.dot_general(zi_hi, zj_hi, dn,
                             preferred_element_type=jnp.float32)
    logits += lax.dot_general(zi_hi, zj_lo, dn,
                              preferred_element_type=jnp.float32)
    logits += lax.dot_general(zi_lo, zj_hi, dn,
                              preferred_element_type=jnp.float32)
    o_ref[...] = 1.0 / (1.0 + jnp.exp(-logits))


def _decoder_tc(z):
    grid = (pl.cdiv(N, BD_I), pl.cdiv(N, BD_J))
    return pl.pallas_call(
        _decoder_body,
        grid=grid,
        in_specs=[
            pl.BlockSpec((BD_I, D), lambda i, j: (i, _i0())),
            pl.BlockSpec((BD_J, D), lambda i, j: (j, _i0())),
        ],
        out_specs=pl.BlockSpec((BD_I, BD_J), lambda i, j: (i, j)),
        out_shape=jax.ShapeDtypeStruct((N, N), jnp.float32),
        compiler_params=pltpu.CompilerParams(
            dimension_semantics=("parallel", "parallel"),
        ),
    )(z, z)


def kernel(x, adj, W_rel, b_rel, W_root):
    x = x.astype(jnp.float32)
    src = adj[0].astype(jnp.int32)
    dst = adj[1].astype(jnp.int32)
    pad = E_PAD - E
    src = jnp.concatenate([src, jnp.zeros((pad,), jnp.int32)])
    dst = jnp.concatenate([dst, jnp.full((pad,), N_PAD - 1, jnp.int32)])

    w_rel_t = W_rel.astype(jnp.float32).T
    w_root_t = W_root.astype(jnp.float32).T
    b2d = b_rel.astype(jnp.float32).reshape(1, D)

    p0, p1 = _sc_scatter(x, src, dst)
    h1 = _conv_tc(p0, p1, x, w_rel_t, w_root_t, b2d, relu=True)
    q0, q1 = _sc_scatter(h1, src, dst)
    x2 = _conv_tc(q0, q1, h1, w_rel_t, w_root_t, b2d, relu=False)
    z_pad = jnp.pad(x2, ((0, N_PAD - N), (0, 0)))
    A = _decoder_tc(z_pad)
    return (A, x2)


# speedup vs baseline: 1.3047x; 1.0909x over previous
import functools

import jax
import jax.numpy as jnp
from jax import lax
from jax.experimental import pallas as pl
from jax.experimental.pallas import tpu as pltpu
from jax.experimental.pallas import tpu_sc as plsc

N = 10000
D = 128
E = 320000

NC = 2
NS = 16
NW = NC * NS

CH = 128
TOT_CHUNKS = 2560
E_PAD = TOT_CHUNKS * CH
K0 = 152
K1 = 8
assert NS * (K0 + K1) == TOT_CHUNKS and K0 % 2 == 0 and K1 % 2 == 0

N_PAD = 10240
RPS = N_PAD // NS

def _i0():
    return jnp.int32(0)


BM = 1000
BD_I = 1024
BD_J = 1024


def _sc_scatter_body(h_hbm, src_hbm, dst_hbm, out0, out1,
                     srcv0, srcv1, dstv0, dstv1, rows0, rows1, acc_sh,
                     sis0, sis1, sid0, sid1, sg0, sg1):
    cid = lax.axis_index("c")
    sid = lax.axis_index("s")

    kk = jnp.where(cid == 0, jnp.int32(K0), jnp.int32(K1))
    base = jnp.where(cid == 0, sid * jnp.int32(K0),
                     jnp.int32(NS * K0) + sid * jnp.int32(K1))

    zv = jnp.zeros((16,), jnp.float32)

    def zrow(r, carry):
        for c8 in range(D // 16):
            rows0[r, pl.ds(jnp.int32(c8 * 16), 16)] = zv
        return carry

    lax.fori_loop(jnp.int32(0), jnp.int32(CH), zrow, jnp.int32(0))
    for s5 in range(RPS // CH):
        pltpu.sync_copy(
            rows0, acc_sh.at[pl.ds(sid * RPS + jnp.int32(s5 * CH), CH)])

    plsc.subcore_barrier()

    def edge_sl(c):
        return pl.ds((base + c) * jnp.int32(CH), CH)

    si = (sis0, sis1)
    sd = (sid0, sid1)
    sg = (sg0, sg1)
    srcb = (srcv0, srcv1)
    dstb = (dstv0, dstv1)
    rows = (rows0, rows1)

    def idx_dma(c, p):
        pltpu.async_copy(src_hbm.at[edge_sl(c)], srcb[p], si[p])
        pltpu.async_copy(dst_hbm.at[edge_sl(c)], dstb[p], sd[p])

    def wait_idx(p):
        z = pl.ds(jnp.int32(0), CH)
        pltpu.make_async_copy(src_hbm.at[z], srcb[p], si[p]).wait()
        pltpu.make_async_copy(dst_hbm.at[z], dstb[p], sd[p]).wait()

    def gather(c, p):
        pltpu.async_copy(h_hbm.at[srcb[p]], rows[p], sg[p])

    def wait_gather(p):
        pltpu.make_async_copy(h_hbm.at[srcb[p]], rows[p], sg[p]).wait()

    def scatter(p):
        pltpu.sync_copy(rows[p], acc_sh.at[dstb[p]], add=True)

    idx_dma(jnp.int32(0), 0)
    idx_dma(jnp.int32(1), 1)
    wait_idx(0)
    gather(jnp.int32(0), 0)

    def step(i, carry):
        c = i * jnp.int32(2)
        wait_idx(1)
        gather(c + 1, 1)
        wait_gather(0)
        scatter(0)
        idx_dma(c + 2, 0)
        wait_idx(0)
        gather(c + 2, 0)
        wait_gather(1)
        scatter(1)
        idx_dma(c + 3, 1)
        return carry

    lax.fori_loop(jnp.int32(0), (kk - 2) // 2, step, jnp.int32(0))
    wait_idx(1)
    gather(kk - 1, 1)
    wait_gather(0)
    scatter(0)
    wait_gather(1)
    scatter(1)

    plsc.subcore_barrier()

    osl = pl.ds(sid * RPS, RPS)

    @pl.when(cid == 0)
    def _():
        pltpu.sync_copy(acc_sh.at[osl], out0.at[osl])

    @pl.when(cid == 1)
    def _():
        pltpu.sync_copy(acc_sh.at[osl], out1.at[osl])


def _sc_scatter(h, src, dst):
    mesh = plsc.VectorSubcoreMesh(core_axis_name="c", subcore_axis_name="s")
    f = functools.partial(
        pl.kernel,
        out_type=(
            jax.ShapeDtypeStruct((N_PAD, D), jnp.float32),
            jax.ShapeDtypeStruct((N_PAD, D), jnp.float32),
        ),
        mesh=mesh,
        scratch_types=[
            pltpu.VMEM((CH,), jnp.int32),
            pltpu.VMEM((CH,), jnp.int32),
            pltpu.VMEM((CH,), jnp.int32),
            pltpu.VMEM((CH,), jnp.int32),
            pltpu.VMEM((CH, D), jnp.float32),
            pltpu.VMEM((CH, D), jnp.float32),
            pltpu.VMEM_SHARED((N_PAD, D), jnp.float32),
            pltpu.SemaphoreType.DMA,
            pltpu.SemaphoreType.DMA,
            pltpu.SemaphoreType.DMA,
            pltpu.SemaphoreType.DMA,
            pltpu.SemaphoreType.DMA,
            pltpu.SemaphoreType.DMA,
        ],
    )(_sc_scatter_body)
    return f(h, src, dst)


def _conv_body(p0_ref, p1_ref, x_ref, wr_ref, wt_ref, b_ref, o_ref, *, relu):
    agg = p0_ref[...] + p1_ref[...]
    y = (
        jnp.dot(agg, wr_ref[...], preferred_element_type=jnp.float32,
                precision=lax.Precision.HIGHEST)
        + b_ref[...]
        + jnp.dot(x_ref[...], wt_ref[...], preferred_element_type=jnp.float32,
                  precision=lax.Precision.HIGHEST)
    )
    if relu:
        y = jnp.maximum(y, 0.0)
    o_ref[...] = y


def _conv_tc(p0, p1, x, w_rel_t, w_root_t, b2d, relu):
    grid = (N // BM,)
    return pl.pallas_call(
        functools.partial(_conv_body, relu=relu),
        grid=grid,
        in_specs=[
            pl.BlockSpec((BM, D), lambda i: (i, _i0())),
            pl.BlockSpec((BM, D), lambda i: (i, _i0())),
            pl.BlockSpec((BM, D), lambda i: (i, _i0())),
            pl.BlockSpec((D, D), lambda i: (_i0(), _i0())),
            pl.BlockSpec((D, D), lambda i: (_i0(), _i0())),
            pl.BlockSpec((1, D), lambda i: (_i0(), _i0())),
        ],
        out_specs=pl.BlockSpec((BM, D), lambda i: (i, _i0())),
        out_shape=jax.ShapeDtypeStruct((N, D), jnp.float32),
    )(p0, p1, x, w_rel_t, w_root_t, b2d)


def _decoder_body(zi_ref, zj_ref, o_ref):
    zi = zi_ref[...]
    zj = zj_ref[...]
    zi_hi = zi.astype(jnp.bfloat16)
    zi_lo = (zi - zi_hi.astype(jnp.float32)).astype(jnp.bfloat16)
    zj_hi = zj.astype(jnp.bfloat16)
    zj_lo = (zj - zj_hi.astype(jnp.float32)).astype(jnp.bfloat16)
    dn = (((1,), (1,)), ((), ()))
    logits = lax.dot_general(zi_hi, zj_hi, dn,
                             preferred_element_type=jnp.float32)
    logits += lax.dot_general(zi_hi, zj_lo, dn,
                              preferred_element_type=jnp.float32)
    logits += lax.dot_general(zi_lo, zj_hi, dn,
                              preferred_element_type=jnp.float32)
    o_ref[...] = 1.0 / (1.0 + jnp.exp(-logits))


def _decoder_tc(z):
    grid = (pl.cdiv(N, BD_I), pl.cdiv(N, BD_J))
    return pl.pallas_call(
        _decoder_body,
        grid=grid,
        in_specs=[
            pl.BlockSpec((BD_I, D), lambda i, j: (i, _i0())),
            pl.BlockSpec((BD_J, D), lambda i, j: (j, _i0())),
        ],
        out_specs=pl.BlockSpec((BD_I, BD_J), lambda i, j: (i, j)),
        out_shape=jax.ShapeDtypeStruct((N, N), jnp.float32),
        compiler_params=pltpu.CompilerParams(
            dimension_semantics=("parallel", "parallel"),
        ),
    )(z, z)


def kernel(x, adj, W_rel, b_rel, W_root):
    x = x.astype(jnp.float32)
    src = adj[0].astype(jnp.int32)
    dst = adj[1].astype(jnp.int32)
    pad = E_PAD - E
    src = jnp.concatenate([src, jnp.zeros((pad,), jnp.int32)])
    dst = jnp.concatenate([dst, jnp.full((pad,), N_PAD - 1, jnp.int32)])

    w_rel_t = W_rel.astype(jnp.float32).T
    w_root_t = W_root.astype(jnp.float32).T
    b2d = b_rel.astype(jnp.float32).reshape(1, D)

    p0, p1 = _sc_scatter(x, src, dst)
    h1 = _conv_tc(p0, p1, x, w_rel_t, w_root_t, b2d, relu=True)
    q0, q1 = _sc_scatter(h1, src, dst)
    x2 = _conv_tc(q0, q1, h1, w_rel_t, w_root_t, b2d, relu=False)
    z_pad = jnp.pad(x2, ((0, N_PAD - N), (0, 0)))
    A = _decoder_tc(z_pad)
    return (A, x2)


# speedup vs baseline: 1.3058x; 1.0008x over previous
import functools

import jax
import jax.numpy as jnp
from jax import lax
from jax.experimental import pallas as pl
from jax.experimental.pallas import tpu as pltpu
from jax.experimental.pallas import tpu_sc as plsc

N = 10000
D = 128
E = 320000

NC = 2
NS = 16
NW = NC * NS

CH = 128
TOT_CHUNKS = 2560
E_PAD = TOT_CHUNKS * CH
K0 = 152
K1 = 8
assert NS * (K0 + K1) == TOT_CHUNKS and K0 % 2 == 0 and K1 % 2 == 0

N_PAD = 10240
RPS = N_PAD // NS

def _i0():
    return jnp.int32(0)


BM = 1000
BD_I = 1024
BD_J = 1024


def _sc_scatter_body(h_hbm, src_hbm, dst_hbm, out0, out1,
                     srcv0, srcv1, dstv0, dstv1, rows0, rows1, acc_sh,
                     sis0, sis1, sid0, sid1, sg0, sg1):
    cid = lax.axis_index("c")
    sid = lax.axis_index("s")

    kk = jnp.where(cid == 0, jnp.int32(K0), jnp.int32(K1))
    base = jnp.where(cid == 0, sid * jnp.int32(K0),
                     jnp.int32(NS * K0) + sid * jnp.int32(K1))

    zv = jnp.zeros((16,), jnp.float32)

    def zrow(r, carry):
        for c8 in range(D // 16):
            rows0[r, pl.ds(jnp.int32(c8 * 16), 16)] = zv
        return carry

    lax.fori_loop(jnp.int32(0), jnp.int32(CH), zrow, jnp.int32(0))
    for s5 in range(RPS // CH):
        pltpu.sync_copy(
            rows0, acc_sh.at[pl.ds(sid * RPS + jnp.int32(s5 * CH), CH)])

    plsc.subcore_barrier()

    def edge_sl(c):
        return pl.ds((base + c) * jnp.int32(CH), CH)

    si = (sis0, sis1)
    sd = (sid0, sid1)
    sg = (sg0, sg1)
    srcb = (srcv0, srcv1)
    dstb = (dstv0, dstv1)
    rows = (rows0, rows1)

    def idx_dma(c, p):
        pltpu.async_copy(src_hbm.at[edge_sl(c)], srcb[p], si[p])
        pltpu.async_copy(dst_hbm.at[edge_sl(c)], dstb[p], sd[p])

    def wait_idx(p):
        z = pl.ds(jnp.int32(0), CH)
        pltpu.make_async_copy(src_hbm.at[z], srcb[p], si[p]).wait()
        pltpu.make_async_copy(dst_hbm.at[z], dstb[p], sd[p]).wait()

    def gather(c, p):
        pltpu.async_copy(h_hbm.at[srcb[p]], rows[p], sg[p])

    def wait_gather(p):
        pltpu.make_async_copy(h_hbm.at[srcb[p]], rows[p], sg[p]).wait()

    def scatter(p):
        pltpu.sync_copy(rows[p], acc_sh.at[dstb[p]], add=True)

    idx_dma(jnp.int32(0), 0)
    idx_dma(jnp.int32(1), 1)
    wait_idx(0)
    gather(jnp.int32(0), 0)

    def step(i, carry):
        c = i * jnp.int32(2)
        wait_idx(1)
        gather(c + 1, 1)
        wait_gather(0)
        scatter(0)
        idx_dma(c + 2, 0)
        wait_idx(0)
        gather(c + 2, 0)
        wait_gather(1)
        scatter(1)
        idx_dma(c + 3, 1)
        return carry

    lax.fori_loop(jnp.int32(0), (kk - 2) // 2, step, jnp.int32(0))
    wait_idx(1)
    gather(kk - 1, 1)
    wait_gather(0)
    scatter(0)
    wait_gather(1)
    scatter(1)

    plsc.subcore_barrier()

    rq = RPS // 4
    osems = (sis0, sis1, sid0, sid1)

    def osl_k(k):
        return pl.ds(sid * RPS + jnp.int32(k * rq), rq)

    @pl.when(cid == 0)
    def _():
        for k in range(4):
            pltpu.async_copy(acc_sh.at[osl_k(k)], out0.at[osl_k(k)], osems[k])
        for k in range(4):
            pltpu.make_async_copy(acc_sh.at[osl_k(k)], out0.at[osl_k(k)],
                                  osems[k]).wait()

    @pl.when(cid == 1)
    def _():
        for k in range(4):
            pltpu.async_copy(acc_sh.at[osl_k(k)], out1.at[osl_k(k)], osems[k])
        for k in range(4):
            pltpu.make_async_copy(acc_sh.at[osl_k(k)], out1.at[osl_k(k)],
                                  osems[k]).wait()


def _sc_scatter(h, src, dst):
    mesh = plsc.VectorSubcoreMesh(core_axis_name="c", subcore_axis_name="s")
    f = functools.partial(
        pl.kernel,
        out_type=(
            jax.ShapeDtypeStruct((N_PAD, D), jnp.float32),
            jax.ShapeDtypeStruct((N_PAD, D), jnp.float32),
        ),
        mesh=mesh,
        scratch_types=[
            pltpu.VMEM((CH,), jnp.int32),
            pltpu.VMEM((CH,), jnp.int32),
            pltpu.VMEM((CH,), jnp.int32),
            pltpu.VMEM((CH,), jnp.int32),
            pltpu.VMEM((CH, D), jnp.float32),
            pltpu.VMEM((CH, D), jnp.float32),
            pltpu.VMEM_SHARED((N_PAD, D), jnp.float32),
            pltpu.SemaphoreType.DMA,
            pltpu.SemaphoreType.DMA,
            pltpu.SemaphoreType.DMA,
            pltpu.SemaphoreType.DMA,
            pltpu.SemaphoreType.DMA,
            pltpu.SemaphoreType.DMA,
        ],
    )(_sc_scatter_body)
    return f(h, src, dst)


def _conv_body(p0_ref, p1_ref, x_ref, wr_ref, wt_ref, b_ref, o_ref, *, relu):
    agg = p0_ref[...] + p1_ref[...]
    y = (
        jnp.dot(agg, wr_ref[...], preferred_element_type=jnp.float32,
                precision=lax.Precision.HIGHEST)
        + b_ref[...]
        + jnp.dot(x_ref[...], wt_ref[...], preferred_element_type=jnp.float32,
                  precision=lax.Precision.HIGHEST)
    )
    if relu:
        y = jnp.maximum(y, 0.0)
    o_ref[...] = y


def _conv_tc(p0, p1, x, w_rel_t, w_root_t, b2d, relu):
    grid = (N // BM,)
    return pl.pallas_call(
        functools.partial(_conv_body, relu=relu),
        grid=grid,
        in_specs=[
            pl.BlockSpec((BM, D), lambda i: (i, _i0())),
            pl.BlockSpec((BM, D), lambda i: (i, _i0())),
            pl.BlockSpec((BM, D), lambda i: (i, _i0())),
            pl.BlockSpec((D, D), lambda i: (_i0(), _i0())),
            pl.BlockSpec((D, D), lambda i: (_i0(), _i0())),
            pl.BlockSpec((1, D), lambda i: (_i0(), _i0())),
        ],
        out_specs=pl.BlockSpec((BM, D), lambda i: (i, _i0())),
        out_shape=jax.ShapeDtypeStruct((N, D), jnp.float32),
    )(p0, p1, x, w_rel_t, w_root_t, b2d)


def _decoder_body(zi_ref, zj_ref, o_ref):
    zi = zi_ref[...]
    zj = zj_ref[...]
    zi_hi = zi.astype(jnp.bfloat16)
    zi_lo = (zi - zi_hi.astype(jnp.float32)).astype(jnp.bfloat16)
    zj_hi = zj.astype(jnp.bfloat16)
    zj_lo = (zj - zj_hi.astype(jnp.float32)).astype(jnp.bfloat16)
    dn = (((1,), (1,)), ((), ()))
    logits = lax.dot_general(zi_hi, zj_hi, dn,
                             preferred_element_type=jnp.float32)
    logits += lax.dot_general(zi_hi, zj_lo, dn,
                              preferred_element_type=jnp.float32)
    logits += lax.dot_general(zi_lo, zj_hi, dn,
                              preferred_element_type=jnp.float32)
    o_ref[...] = 1.0 / (1.0 + jnp.exp(-logits))


def _decoder_tc(z):
    grid = (pl.cdiv(N, BD_I), pl.cdiv(N, BD_J))
    return pl.pallas_call(
        _decoder_body,
        grid=grid,
        in_specs=[
            pl.BlockSpec((BD_I, D), lambda i, j: (i, _i0())),
            pl.BlockSpec((BD_J, D), lambda i, j: (j, _i0())),
        ],
        out_specs=pl.BlockSpec((BD_I, BD_J), lambda i, j: (i, j)),
        out_shape=jax.ShapeDtypeStruct((N, N), jnp.float32),
        compiler_params=pltpu.CompilerParams(
            dimension_semantics=("parallel", "parallel"),
        ),
    )(z, z)


def kernel(x, adj, W_rel, b_rel, W_root):
    x = x.astype(jnp.float32)
    src = adj[0].astype(jnp.int32)
    dst = adj[1].astype(jnp.int32)
    pad = E_PAD - E
    src = jnp.concatenate([src, jnp.zeros((pad,), jnp.int32)])
    dst = jnp.concatenate([dst, jnp.full((pad,), N_PAD - 1, jnp.int32)])

    w_rel_t = W_rel.astype(jnp.float32).T
    w_root_t = W_root.astype(jnp.float32).T
    b2d = b_rel.astype(jnp.float32).reshape(1, D)

    p0, p1 = _sc_scatter(x, src, dst)
    h1 = _conv_tc(p0, p1, x, w_rel_t, w_root_t, b2d, relu=True)
    q0, q1 = _sc_scatter(h1, src, dst)
    x2 = _conv_tc(q0, q1, h1, w_rel_t, w_root_t, b2d, relu=False)
    z_pad = jnp.pad(x2, ((0, N_PAD - N), (0, 0)))
    A = _decoder_tc(z_pad)
    return (A, x2)
